# baseline (speedup 1.0000x reference)
import functools

import jax
import jax.numpy as jnp
from jax import lax
from jax.experimental import pallas as pl
from jax.experimental.pallas import tpu as pltpu
from jax.experimental.pallas import tpu_sc as plsc

N = 10000
D = 128
DE = 16
E = 320000
CH = 128
GC = 2
CT = 2560
EPAD = CT * CH
NG = CT // GC
NACC = 10112
ZROWS = NACC // 16
G1 = NG // 16
G2 = NG // 32

_f32 = jnp.float32


def _sweep(tbl_hbm, idx2_hbm, acc, idxA, idxB, rows, gsem, ssem,
           slab, gbase, ngrp, linear_src=None):
    dummy = tbl_hbm.at[pl.ds(0, CH)] if linear_src is None else \
        linear_src.at[pl.ds(0, CH)]

    def issue(buf_idx, b, g):
        if linear_src is None:
            pltpu.async_copy(tbl_hbm.at[buf_idx.at[b]], rows[b], gsem[b])
        else:
            chunk = g * GC + b
            pltpu.async_copy(linear_src.at[pl.ds(chunk * CH, CH)],
                             rows[b], gsem[b])

    pltpu.sync_copy(idx2_hbm.at[slab, gbase], idxA)
    for b in range(GC):
        issue(idxA, b, gbase)

    def half(g, X, Y):
        pltpu.sync_copy(idx2_hbm.at[slab, g + 1], Y)
        for b in range(GC):
            pltpu.make_async_copy(dummy, rows[b], gsem[b]).wait()
            pltpu.async_copy(rows[b], acc.at[X.at[GC + b]], ssem[b], add=True)
        for b in range(GC):
            pltpu.make_async_copy(rows[b], acc.at[pl.ds(0, CH)], ssem[b]).wait()
            issue(Y, b, g + 1)

    def body(i, carry):
        g = gbase + 2 * i
        half(g, idxA, idxB)
        half(g + 1, idxB, idxA)
        return carry

    lax.fori_loop(0, ngrp // 2, body, 0)
    for b in range(GC):
        pltpu.make_async_copy(dummy, rows[b], gsem[b]).wait()


def _sc_pass1(xh_hbm, ef_hbm, idx2_hbm, z128_hbm,
              sxh_out, se_out,
              acc128, idxA, idxB, r0b, r1b,
              gs0, gs1, ss0, ss1):
    c = lax.axis_index("c")
    s = lax.axis_index("s")
    rows = [r0b, r1b]
    gsem = [gs0, gs1]
    ssem = [ss0, ss1]
    r0 = s * ZROWS
    pltpu.sync_copy(z128_hbm.at[pl.ds(r0, ZROWS)], acc128.at[pl.ds(r0, ZROWS)])
    plsc.subcore_barrier()

    _sweep(xh_hbm, idx2_hbm, acc128, idxA, idxB, rows, gsem, ssem,
           slab=c, gbase=s * G1, ngrp=G1)
    plsc.subcore_barrier()
    pltpu.sync_copy(acc128.at[pl.ds(r0, ZROWS)], sxh_out.at[c, pl.ds(r0, ZROWS)])

    pltpu.sync_copy(z128_hbm.at[pl.ds(r0, ZROWS)], acc128.at[pl.ds(r0, ZROWS)])
    plsc.subcore_barrier()
    _sweep(xh_hbm, idx2_hbm, acc128, idxA, idxB, rows, gsem, ssem,
           slab=0, gbase=(c * 16 + s) * G2, ngrp=G2, linear_src=ef_hbm)
    plsc.subcore_barrier()
    pltpu.sync_copy(acc128.at[pl.ds(r0, ZROWS)], se_out.at[c, pl.ds(r0, ZROWS)])


def _sc_pass2(g_hbm, idx2_hbm, z128_hbm,
              sg_out,
              acc128, idxA, idxB, r0b, r1b,
              gs0, gs1, ss0, ss1):
    c = lax.axis_index("c")
    s = lax.axis_index("s")
    rows = [r0b, r1b]
    gsem = [gs0, gs1]
    ssem = [ss0, ss1]
    r0 = s * ZROWS
    pltpu.sync_copy(z128_hbm.at[pl.ds(r0, ZROWS)], acc128.at[pl.ds(r0, ZROWS)])
    plsc.subcore_barrier()

    _sweep(g_hbm, idx2_hbm, acc128, idxA, idxB, rows, gsem, ssem,
           slab=0, gbase=(c * 16 + s) * G2, ngrp=G2)
    plsc.subcore_barrier()
    pltpu.sync_copy(acc128.at[pl.ds(r0, ZROWS)], sg_out.at[c, pl.ds(r0, ZROWS)])


_SC_SCRATCH = [
    pltpu.VMEM_SHARED((NACC, D), _f32),
    pltpu.VMEM((2 * GC, CH), jnp.int32),
    pltpu.VMEM((2 * GC, CH), jnp.int32),
    pltpu.VMEM((CH, D), _f32),
    pltpu.VMEM((CH, D), _f32),
    pltpu.SemaphoreType.DMA,
    pltpu.SemaphoreType.DMA,
    pltpu.SemaphoreType.DMA,
    pltpu.SemaphoreType.DMA,
]


@functools.lru_cache(maxsize=1)
def _build_sc_kernels():
    mesh = plsc.VectorSubcoreMesh(core_axis_name="c", subcore_axis_name="s",
                                  num_cores=2, num_subcores=16)
    pass1 = functools.partial(
        pl.kernel,
        out_type=(
            jax.ShapeDtypeStruct((2, NACC, D), _f32),
            jax.ShapeDtypeStruct((2, NACC, D), _f32),
        ),
        mesh=mesh,
        scratch_types=list(_SC_SCRATCH),
    )(_sc_pass1)
    pass2 = functools.partial(
        pl.kernel,
        out_type=jax.ShapeDtypeStruct((2, NACC, D), _f32),
        mesh=mesh,
        scratch_types=list(_SC_SCRATCH),
    )(_sc_pass2)
    return pass1, pass2


BR = 400
GRID = N // BR


def _tc1_body(x, h, sx, sh, sea, seb, w1, w3, w5, w2, w4, wshh, bv,
              z_o, g_o, t2_o):
    se = sea[...] + seb[...]
    u = (jnp.dot(x[...], w1[...], preferred_element_type=_f32)
         + jnp.dot(sx[...], w3[...], preferred_element_type=_f32)
         + jnp.dot(se, w5[...], preferred_element_type=_f32)
         + bv[...])
    u2 = (u[:, 0:2 * D]
          + jnp.dot(h[...], w2[...], preferred_element_type=_f32)
          + jnp.dot(sh[...], w4[...], preferred_element_type=_f32))
    z = jax.nn.sigmoid(u2[:, 0:D])
    r = jax.nn.sigmoid(u2[:, D:2 * D])
    g = h[...] * r
    z_o[...] = z
    g_o[...] = g
    t2_o[...] = u[:, 2 * D:3 * D] + jnp.dot(g, wshh[...], preferred_element_type=_f32)


_row_spec = pl.BlockSpec((BR, D), lambda i: (i, 0))

_tc1 = pl.pallas_call(
    _tc1_body,
    grid=(GRID,),
    in_specs=[
        _row_spec,
        _row_spec,
        _row_spec,
        _row_spec,
        _row_spec,
        _row_spec,
        pl.BlockSpec((D, 3 * D), lambda i: (0, 0)),
        pl.BlockSpec((D, 3 * D), lambda i: (0, 0)),
        pl.BlockSpec((D, 3 * D), lambda i: (0, 0)),
        pl.BlockSpec((D, 2 * D), lambda i: (0, 0)),
        pl.BlockSpec((D, 2 * D), lambda i: (0, 0)),
        pl.BlockSpec((D, D), lambda i: (0, 0)),
        pl.BlockSpec((1, 3 * D), lambda i: (0, 0)),
    ],
    out_specs=[_row_spec, _row_spec, _row_spec],
    out_shape=[
        jax.ShapeDtypeStruct((N, D), _f32),
        jax.ShapeDtypeStruct((N, D), _f32),
        jax.ShapeDtypeStruct((N, D), _f32),
    ],
)


def _tc2_body(z, h, t2, sga, sgb, wmhh, out):
    sg = sga[...] + sgb[...]
    ht = jnp.tanh(t2[...] + jnp.dot(sg, wmhh[...], preferred_element_type=_f32))
    out[...] = z[...] * h[...] + (1.0 - z[...]) * ht


_tc2 = pl.pallas_call(
    _tc2_body,
    grid=(GRID,),
    in_specs=[
        _row_spec,
        _row_spec,
        _row_spec,
        _row_spec,
        _row_spec,
        pl.BlockSpec((D, D), lambda i: (0, 0)),
    ],
    out_specs=_row_spec,
    out_shape=jax.ShapeDtypeStruct((N, D), _f32),
)


def kernel(node_feature, edge_index, edge_feature, H,
           Wm_xz, Ws_xz, b_xz, Wm_hz, Ws_hz, b_hz,
           Wm_xr, Ws_xr, b_xr, Wm_hr, Ws_hr, b_hr,
           Wm_xh, Ws_xh, b_xh, Wm_hh, Ws_hh, b_hh):
    src = edge_index[0]
    dst = edge_index[1]
    pad = EPAD - E
    srcp = jnp.concatenate([src, jnp.zeros((pad,), jnp.int32)])
    dstp = jnp.concatenate([dst, jnp.full((pad,), N, jnp.int32)])
    s4 = srcp.reshape(NG, GC, CH)
    d4 = dstp.reshape(NG, GC, CH)
    idx2 = jnp.stack([jnp.concatenate([s4, d4], axis=1),
                      jnp.concatenate([s4 + N, d4], axis=1)])
    padgrp = jnp.concatenate([jnp.zeros((2, 1, GC, CH), jnp.int32),
                              jnp.full((2, 1, GC, CH), N, jnp.int32)], axis=2)
    idx2 = jnp.concatenate([idx2, padgrp], axis=1)
    efp = jnp.pad(edge_feature, ((0, pad + GC * CH), (0, D - DE)))
    xh = jnp.concatenate([node_feature, H], axis=0)
    z128 = jnp.zeros((NACC, D), _f32)

    sc1, sc2 = _build_sc_kernels()
    sxh, se2 = sc1(xh, efp, idx2, z128)

    w1 = jnp.concatenate([Ws_xz, Ws_xr, Ws_xh], axis=1)
    w3 = jnp.concatenate([Wm_xz[:D], Wm_xr[:D], Wm_xh[:D]], axis=1)
    w5 = jnp.concatenate([Wm_xz[D:] + Wm_hz[D:],
                          Wm_xr[D:] + Wm_hr[D:],
                          Wm_xh[D:] + Wm_hh[D:]], axis=1)
    w5 = jnp.pad(w5, ((0, D - DE), (0, 0)))
    w2 = jnp.concatenate([Ws_hz, Ws_hr], axis=1)
    w4 = jnp.concatenate([Wm_hz[:D], Wm_hr[:D]], axis=1)
    bv = jnp.concatenate([b_xz + b_hz, b_xr + b_hr, b_xh + b_hh]).reshape(1, 3 * D)

    z, g, t2 = _tc1(node_feature, H, sxh[0], sxh[1], se2[0], se2[1],
                    w1, w3, w5, w2, w4, Ws_hh, bv)

    sg2 = sc2(g, idx2, z128)

    return _tc2(z, H, t2, sg2[0], sg2[1], Wm_hh[:D])


# speedup vs baseline: 1.1343x; 1.1343x over previous
import functools

import jax
import jax.numpy as jnp
from jax import lax
from jax.experimental import pallas as pl
from jax.experimental.pallas import tpu as pltpu
from jax.experimental.pallas import tpu_sc as plsc

N = 10000
D = 128
DE = 16
E = 320000
CH = 128
GC = 2
CT = 2560
EPAD = CT * CH
NG = CT // GC
NACC = 10112
ZROWS = NACC // 16
G1 = NG // 16
G2 = NG // 32

_f32 = jnp.float32


def _sweep(tbl_hbm, idx2_hbm, acc, idxA, idxB, rows, gsem, ssem,
           slab, gbase, ngrp, stride=1, linear_src=None):
    dummy = tbl_hbm.at[pl.ds(0, CH)] if linear_src is None else \
        linear_src.at[pl.ds(0, CH)]

    def issue(buf_idx, b, g):
        if linear_src is None:
            pltpu.async_copy(tbl_hbm.at[buf_idx.at[b]], rows[b], gsem[b])
        else:
            chunk = g * GC + b
            pltpu.async_copy(linear_src.at[pl.ds(chunk * CH, CH)],
                             rows[b], gsem[b])

    st = stride

    pltpu.sync_copy(idx2_hbm.at[slab, gbase], idxA)
    for b in range(GC):
        issue(idxA, b, gbase)

    def half(g, X, Y):
        pltpu.sync_copy(idx2_hbm.at[slab, g + st], Y)
        for b in range(GC):
            pltpu.make_async_copy(dummy, rows[b], gsem[b]).wait()
            pltpu.async_copy(rows[b], acc.at[X.at[GC + b]], ssem[b],
                             add=True).wait()
            issue(Y, b, g + st)

    def body(i, carry):
        g = gbase + 2 * st * i
        half(g, idxA, idxB)
        half(g + st, idxB, idxA)
        return carry

    lax.fori_loop(0, ngrp // 2, body, 0)
    for b in range(GC):
        pltpu.make_async_copy(dummy, rows[b], gsem[b]).wait()


def _sc_pass1(xh_hbm, ef_hbm, idx2_hbm, z128_hbm,
              sxh_out, se_out,
              acc128, idxA, idxB, r0b, r1b,
              gs0, gs1, ss0, ss1):
    c = lax.axis_index("c")
    s = lax.axis_index("s")
    rows = [r0b, r1b]
    gsem = [gs0, gs1]
    ssem = [ss0, ss1]
    r0 = s * ZROWS
    pltpu.sync_copy(z128_hbm.at[pl.ds(r0, ZROWS)], acc128.at[pl.ds(r0, ZROWS)])
    plsc.subcore_barrier()

    _sweep(xh_hbm, idx2_hbm, acc128, idxA, idxB, rows, gsem, ssem,
           slab=c, gbase=s * G1, ngrp=G1)
    plsc.subcore_barrier()
    pltpu.sync_copy(acc128.at[pl.ds(r0, ZROWS)], sxh_out.at[c, pl.ds(r0, ZROWS)])

    pltpu.sync_copy(z128_hbm.at[pl.ds(r0, ZROWS)], acc128.at[pl.ds(r0, ZROWS)])
    plsc.subcore_barrier()
    _sweep(xh_hbm, idx2_hbm, acc128, idxA, idxB, rows, gsem, ssem,
           slab=0, gbase=s * 2 * G2 + c, ngrp=G2, stride=2, linear_src=ef_hbm)
    plsc.subcore_barrier()
    pltpu.sync_copy(acc128.at[pl.ds(r0, ZROWS)], se_out.at[c, pl.ds(r0, ZROWS)])


def _sc_pass2(g_hbm, idx2_hbm, z128_hbm,
              sg_out,
              acc128, idxA, idxB, r0b, r1b,
              gs0, gs1, ss0, ss1):
    c = lax.axis_index("c")
    s = lax.axis_index("s")
    rows = [r0b, r1b]
    gsem = [gs0, gs1]
    ssem = [ss0, ss1]
    r0 = s * ZROWS
    pltpu.sync_copy(z128_hbm.at[pl.ds(r0, ZROWS)], acc128.at[pl.ds(r0, ZROWS)])
    plsc.subcore_barrier()

    _sweep(g_hbm, idx2_hbm, acc128, idxA, idxB, rows, gsem, ssem,
           slab=0, gbase=s * 2 * G2 + c, ngrp=G2, stride=2)
    plsc.subcore_barrier()
    pltpu.sync_copy(acc128.at[pl.ds(r0, ZROWS)], sg_out.at[c, pl.ds(r0, ZROWS)])


_SC_SCRATCH = [
    pltpu.VMEM_SHARED((NACC, D), _f32),
    pltpu.VMEM((2 * GC, CH), jnp.int32),
    pltpu.VMEM((2 * GC, CH), jnp.int32),
    pltpu.VMEM((CH, D), _f32),
    pltpu.VMEM((CH, D), _f32),
    pltpu.SemaphoreType.DMA,
    pltpu.SemaphoreType.DMA,
    pltpu.SemaphoreType.DMA,
    pltpu.SemaphoreType.DMA,
]


@functools.lru_cache(maxsize=1)
def _build_sc_kernels():
    mesh = plsc.VectorSubcoreMesh(core_axis_name="c", subcore_axis_name="s",
                                  num_cores=2, num_subcores=16)
    pass1 = functools.partial(
        pl.kernel,
        out_type=(
            jax.ShapeDtypeStruct((2, NACC, D), _f32),
            jax.ShapeDtypeStruct((2, NACC, D), _f32),
        ),
        mesh=mesh,
        scratch_types=list(_SC_SCRATCH),
    )(_sc_pass1)
    pass2 = functools.partial(
        pl.kernel,
        out_type=jax.ShapeDtypeStruct((2, NACC, D), _f32),
        mesh=mesh,
        scratch_types=list(_SC_SCRATCH),
    )(_sc_pass2)
    return pass1, pass2


BR = 400
GRID = N // BR


def _tc1_body(x, h, sx, sh, sea, seb, w1, w3, w5, w2, w4, wshh, bv,
              z_o, g_o, t2_o):
    se = sea[...] + seb[...]
    u = (jnp.dot(x[...], w1[...], preferred_element_type=_f32)
         + jnp.dot(sx[...], w3[...], preferred_element_type=_f32)
         + jnp.dot(se, w5[...], preferred_element_type=_f32)
         + bv[...])
    u2 = (u[:, 0:2 * D]
          + jnp.dot(h[...], w2[...], preferred_element_type=_f32)
          + jnp.dot(sh[...], w4[...], preferred_element_type=_f32))
    z = jax.nn.sigmoid(u2[:, 0:D])
    r = jax.nn.sigmoid(u2[:, D:2 * D])
    g = h[...] * r
    z_o[...] = z
    g_o[...] = g
    t2_o[...] = u[:, 2 * D:3 * D] + jnp.dot(g, wshh[...], preferred_element_type=_f32)


_row_spec = pl.BlockSpec((BR, D), lambda i: (i, 0))

_tc1 = pl.pallas_call(
    _tc1_body,
    grid=(GRID,),
    in_specs=[
        _row_spec,
        _row_spec,
        _row_spec,
        _row_spec,
        _row_spec,
        _row_spec,
        pl.BlockSpec((D, 3 * D), lambda i: (0, 0)),
        pl.BlockSpec((D, 3 * D), lambda i: (0, 0)),
        pl.BlockSpec((D, 3 * D), lambda i: (0, 0)),
        pl.BlockSpec((D, 2 * D), lambda i: (0, 0)),
        pl.BlockSpec((D, 2 * D), lambda i: (0, 0)),
        pl.BlockSpec((D, D), lambda i: (0, 0)),
        pl.BlockSpec((1, 3 * D), lambda i: (0, 0)),
    ],
    out_specs=[_row_spec, _row_spec, _row_spec],
    out_shape=[
        jax.ShapeDtypeStruct((N, D), _f32),
        jax.ShapeDtypeStruct((N, D), _f32),
        jax.ShapeDtypeStruct((N, D), _f32),
    ],
)


def _tc2_body(z, h, t2, sga, sgb, wmhh, out):
    sg = sga[...] + sgb[...]
    ht = jnp.tanh(t2[...] + jnp.dot(sg, wmhh[...], preferred_element_type=_f32))
    out[...] = z[...] * h[...] + (1.0 - z[...]) * ht


_tc2 = pl.pallas_call(
    _tc2_body,
    grid=(GRID,),
    in_specs=[
        _row_spec,
        _row_spec,
        _row_spec,
        _row_spec,
        _row_spec,
        pl.BlockSpec((D, D), lambda i: (0, 0)),
    ],
    out_specs=_row_spec,
    out_shape=jax.ShapeDtypeStruct((N, D), _f32),
)


def kernel(node_feature, edge_index, edge_feature, H,
           Wm_xz, Ws_xz, b_xz, Wm_hz, Ws_hz, b_hz,
           Wm_xr, Ws_xr, b_xr, Wm_hr, Ws_hr, b_hr,
           Wm_xh, Ws_xh, b_xh, Wm_hh, Ws_hh, b_hh):
    src = edge_index[0]
    dst = edge_index[1]
    pad = EPAD - E
    srcp = jnp.concatenate([src, jnp.zeros((pad,), jnp.int32)])
    dstp = jnp.concatenate([dst, jnp.full((pad,), N, jnp.int32)])
    s4 = srcp.reshape(NG, GC, CH)
    d4 = dstp.reshape(NG, GC, CH)
    idx2 = jnp.stack([jnp.concatenate([s4, d4], axis=1),
                      jnp.concatenate([s4 + N, d4], axis=1)])
    padgrp = jnp.concatenate([jnp.zeros((2, 2, GC, CH), jnp.int32),
                              jnp.full((2, 2, GC, CH), N, jnp.int32)], axis=2)
    idx2 = jnp.concatenate([idx2, padgrp], axis=1)
    efp = jnp.pad(edge_feature, ((0, pad + GC * CH), (0, D - DE)))
    xh = jnp.concatenate([node_feature, H], axis=0)
    z128 = jnp.zeros((NACC, D), _f32)

    sc1, sc2 = _build_sc_kernels()
    sxh, se2 = sc1(xh, efp, idx2, z128)

    w1 = jnp.concatenate([Ws_xz, Ws_xr, Ws_xh], axis=1)
    w3 = jnp.concatenate([Wm_xz[:D], Wm_xr[:D], Wm_xh[:D]], axis=1)
    w5 = jnp.concatenate([Wm_xz[D:] + Wm_hz[D:],
                          Wm_xr[D:] + Wm_hr[D:],
                          Wm_xh[D:] + Wm_hh[D:]], axis=1)
    w5 = jnp.pad(w5, ((0, D - DE), (0, 0)))
    w2 = jnp.concatenate([Ws_hz, Ws_hr], axis=1)
    w4 = jnp.concatenate([Wm_hz[:D], Wm_hr[:D]], axis=1)
    bv = jnp.concatenate([b_xz + b_hz, b_xr + b_hr, b_xh + b_hh]).reshape(1, 3 * D)

    z, g, t2 = _tc1(node_feature, H, sxh[0], sxh[1], se2[0], se2[1],
                    w1, w3, w5, w2, w4, Ws_hh, bv)

    sg2 = sc2(g, idx2, z128)

    return _tc2(z, H, t2, sg2[0], sg2[1], Wm_hh[:D])


# speedup vs baseline: 1.1495x; 1.0134x over previous
import functools

import jax
import jax.numpy as jnp
from jax import lax
from jax.experimental import pallas as pl
from jax.experimental.pallas import tpu as pltpu
from jax.experimental.pallas import tpu_sc as plsc

N = 10000
D = 128
DE = 16
E = 320000
CH = 128
GC = 2
CT = 2560
EPAD = CT * CH
NG = CT // GC
NACC = 10112
ZROWS = NACC // 16
G1 = NG // 16
G2 = NG // 32

_f32 = jnp.float32


def _sweep(tbl_hbm, idx2_hbm, acc, idxA, idxB, rows, gsem, ssem,
           slab, gbase, ngrp, stride=1, linear_src=None):
    dummy = tbl_hbm.at[pl.ds(0, CH)] if linear_src is None else \
        linear_src.at[pl.ds(0, CH)]

    def issue(buf_idx, b, g):
        if linear_src is None:
            pltpu.async_copy(tbl_hbm.at[buf_idx.at[b]], rows[b], gsem[b])
        else:
            chunk = g * GC + b
            pltpu.async_copy(linear_src.at[pl.ds(chunk * CH, CH)],
                             rows[b], gsem[b])

    st = stride

    pltpu.sync_copy(idx2_hbm.at[slab, gbase], idxA)
    for b in range(GC):
        issue(idxA, b, gbase)

    def half(g, X, Y):
        pltpu.sync_copy(idx2_hbm.at[slab, g + st], Y)
        for b in range(GC):
            pltpu.make_async_copy(dummy, rows[b], gsem[b]).wait()
            pltpu.async_copy(rows[b], acc.at[X.at[GC + b]], ssem[b],
                             add=True).wait()
            issue(Y, b, g + st)

    def body(i, carry):
        g = gbase + 2 * st * i
        half(g, idxA, idxB)
        half(g + st, idxB, idxA)
        return carry

    lax.fori_loop(0, ngrp // 2, body, 0)
    for b in range(GC):
        pltpu.make_async_copy(dummy, rows[b], gsem[b]).wait()


def _sc_pass1(xh_hbm, ef_hbm, idx2_hbm, z128_hbm,
              sxh_out, se_out,
              acc128, idxA, idxB, r0b, r1b,
              gs0, gs1, ss0, ss1):
    c = lax.axis_index("c")
    s = lax.axis_index("s")
    rows = [r0b, r1b]
    gsem = [gs0, gs1]
    ssem = [ss0, ss1]
    r0 = s * ZROWS
    pltpu.sync_copy(z128_hbm.at[pl.ds(r0, ZROWS)], acc128.at[pl.ds(r0, ZROWS)])
    plsc.subcore_barrier()

    _sweep(xh_hbm, idx2_hbm, acc128, idxA, idxB, rows, gsem, ssem,
           slab=c, gbase=s * G1, ngrp=G1)
    plsc.subcore_barrier()
    pltpu.sync_copy(acc128.at[pl.ds(r0, ZROWS)], sxh_out.at[c, pl.ds(r0, ZROWS)])

    pltpu.sync_copy(z128_hbm.at[pl.ds(r0, ZROWS)], acc128.at[pl.ds(r0, ZROWS)])
    plsc.subcore_barrier()
    _sweep(xh_hbm, idx2_hbm, acc128, idxA, idxB, rows, gsem, ssem,
           slab=0, gbase=s * 2 * G2 + c, ngrp=G2, stride=2, linear_src=ef_hbm)
    plsc.subcore_barrier()
    pltpu.sync_copy(acc128.at[pl.ds(r0, ZROWS)], se_out.at[c, pl.ds(r0, ZROWS)])


def _sc_pass2(g_hbm, idx2_hbm, z128_hbm,
              sg_out,
              acc128, idxA, idxB, r0b, r1b,
              gs0, gs1, ss0, ss1):
    c = lax.axis_index("c")
    s = lax.axis_index("s")
    rows = [r0b, r1b]
    gsem = [gs0, gs1]
    ssem = [ss0, ss1]
    r0 = s * ZROWS
    pltpu.sync_copy(z128_hbm.at[pl.ds(r0, ZROWS)], acc128.at[pl.ds(r0, ZROWS)])
    plsc.subcore_barrier()

    _sweep(g_hbm, idx2_hbm, acc128, idxA, idxB, rows, gsem, ssem,
           slab=0, gbase=s * 2 * G2 + c, ngrp=G2, stride=2)
    plsc.subcore_barrier()
    pltpu.sync_copy(acc128.at[pl.ds(r0, ZROWS)], sg_out.at[c, pl.ds(r0, ZROWS)])


_SC_SCRATCH = [
    pltpu.VMEM_SHARED((NACC, D), _f32),
    pltpu.VMEM((2 * GC, CH), jnp.int32),
    pltpu.VMEM((2 * GC, CH), jnp.int32),
    pltpu.VMEM((CH, D), _f32),
    pltpu.VMEM((CH, D), _f32),
    pltpu.SemaphoreType.DMA,
    pltpu.SemaphoreType.DMA,
    pltpu.SemaphoreType.DMA,
    pltpu.SemaphoreType.DMA,
]


@functools.lru_cache(maxsize=1)
def _build_sc_kernels():
    mesh = plsc.VectorSubcoreMesh(core_axis_name="c", subcore_axis_name="s",
                                  num_cores=2, num_subcores=16)
    pass1 = functools.partial(
        pl.kernel,
        out_type=(
            jax.ShapeDtypeStruct((2, NACC, D), _f32),
            jax.ShapeDtypeStruct((2, NACC, D), _f32),
        ),
        mesh=mesh,
        scratch_types=list(_SC_SCRATCH),
    )(_sc_pass1)
    pass2 = functools.partial(
        pl.kernel,
        out_type=jax.ShapeDtypeStruct((2, NACC, D), _f32),
        mesh=mesh,
        scratch_types=list(_SC_SCRATCH),
    )(_sc_pass2)
    return pass1, pass2


BR = 400
GRID = N // BR


def _tc1_body(x, h, sxh0, sxh1, sea, seb, w1, w3, w5, w2, w4, wshh, bv,
              z_o, g_o, t2_o):
    se = sea[0] + seb[0]
    u = (jnp.dot(x[...], w1[...], preferred_element_type=_f32)
         + jnp.dot(sxh0[0], w3[...], preferred_element_type=_f32)
         + jnp.dot(se, w5[...], preferred_element_type=_f32)
         + bv[...])
    u2 = (u[:, 0:2 * D]
          + jnp.dot(h[...], w2[...], preferred_element_type=_f32)
          + jnp.dot(sxh1[0], w4[...], preferred_element_type=_f32))
    z = jax.nn.sigmoid(u2[:, 0:D])
    r = jax.nn.sigmoid(u2[:, D:2 * D])
    g = h[...] * r
    z_o[...] = z
    g_o[...] = g
    t2_o[...] = u[:, 2 * D:3 * D] + jnp.dot(g, wshh[...], preferred_element_type=_f32)


_row_spec = pl.BlockSpec((BR, D), lambda i: (i, 0))
_slab0_spec = pl.BlockSpec((1, BR, D), lambda i: (0, i, 0))
_slab1_spec = pl.BlockSpec((1, BR, D), lambda i: (1, i, 0))

_tc1 = pl.pallas_call(
    _tc1_body,
    grid=(GRID,),
    in_specs=[
        _row_spec,
        _row_spec,
        _slab0_spec,
        _slab1_spec,
        _slab0_spec,
        _slab1_spec,
        pl.BlockSpec((D, 3 * D), lambda i: (0, 0)),
        pl.BlockSpec((D, 3 * D), lambda i: (0, 0)),
        pl.BlockSpec((D, 3 * D), lambda i: (0, 0)),
        pl.BlockSpec((D, 2 * D), lambda i: (0, 0)),
        pl.BlockSpec((D, 2 * D), lambda i: (0, 0)),
        pl.BlockSpec((D, D), lambda i: (0, 0)),
        pl.BlockSpec((1, 3 * D), lambda i: (0, 0)),
    ],
    out_specs=[_row_spec, _row_spec, _row_spec],
    out_shape=[
        jax.ShapeDtypeStruct((N, D), _f32),
        jax.ShapeDtypeStruct((N, D), _f32),
        jax.ShapeDtypeStruct((N, D), _f32),
    ],
)


def _tc2_body(z, h, t2, sga, sgb, wmhh, out):
    sg = sga[0] + sgb[0]
    ht = jnp.tanh(t2[...] + jnp.dot(sg, wmhh[...], preferred_element_type=_f32))
    out[...] = z[...] * h[...] + (1.0 - z[...]) * ht


_tc2 = pl.pallas_call(
    _tc2_body,
    grid=(GRID,),
    in_specs=[
        _row_spec,
        _row_spec,
        _row_spec,
        _slab0_spec,
        _slab1_spec,
        pl.BlockSpec((D, D), lambda i: (0, 0)),
    ],
    out_specs=_row_spec,
    out_shape=jax.ShapeDtypeStruct((N, D), _f32),
)


def kernel(node_feature, edge_index, edge_feature, H,
           Wm_xz, Ws_xz, b_xz, Wm_hz, Ws_hz, b_hz,
           Wm_xr, Ws_xr, b_xr, Wm_hr, Ws_hr, b_hr,
           Wm_xh, Ws_xh, b_xh, Wm_hh, Ws_hh, b_hh):
    src = edge_index[0]
    dst = edge_index[1]
    pad = EPAD - E
    srcp = jnp.concatenate([src, jnp.zeros((pad,), jnp.int32)])
    dstp = jnp.concatenate([dst, jnp.full((pad,), N, jnp.int32)])
    s4 = srcp.reshape(NG, GC, CH)
    d4 = dstp.reshape(NG, GC, CH)
    idx2 = jnp.stack([jnp.concatenate([s4, d4], axis=1),
                      jnp.concatenate([s4 + N, d4], axis=1)])
    padgrp = jnp.concatenate([jnp.zeros((2, 2, GC, CH), jnp.int32),
                              jnp.full((2, 2, GC, CH), N, jnp.int32)], axis=2)
    idx2 = jnp.concatenate([idx2, padgrp], axis=1)
    efp = jnp.pad(edge_feature, ((0, pad + GC * CH), (0, D - DE)))
    xh = jnp.concatenate([node_feature, H], axis=0)
    z128 = jnp.zeros((NACC, D), _f32)

    sc1, sc2 = _build_sc_kernels()
    sxh, se2 = sc1(xh, efp, idx2, z128)

    w1 = jnp.concatenate([Ws_xz, Ws_xr, Ws_xh], axis=1)
    w3 = jnp.concatenate([Wm_xz[:D], Wm_xr[:D], Wm_xh[:D]], axis=1)
    w5 = jnp.concatenate([Wm_xz[D:] + Wm_hz[D:],
                          Wm_xr[D:] + Wm_hr[D:],
                          Wm_xh[D:] + Wm_hh[D:]], axis=1)
    w5 = jnp.pad(w5, ((0, D - DE), (0, 0)))
    w2 = jnp.concatenate([Ws_hz, Ws_hr], axis=1)
    w4 = jnp.concatenate([Wm_hz[:D], Wm_hr[:D]], axis=1)
    bv = jnp.concatenate([b_xz + b_hz, b_xr + b_hr, b_xh + b_hh]).reshape(1, 3 * D)

    z, g, t2 = _tc1(node_feature, H, sxh, sxh, se2, se2,
                    w1, w3, w5, w2, w4, Ws_hh, bv)

    sg2 = sc2(g, idx2, z128)

    return _tc2(z, H, t2, sg2, sg2, Wm_hh[:D])
